# Initial kernel scaffold; baseline (speedup 1.0000x reference)
import jax
import jax.numpy as jnp
from jax.experimental import pallas as pl


def kernel(x, weight_main, U_all, S_all, V_all, bias, top_k_gates, top_k_indices):
    raise NotImplementedError("write your pallas kernel here")


# speedup vs baseline: 18.4548x; 18.4548x over previous
import functools

import jax
import jax.numpy as jnp
from jax.experimental import pallas as pl

N, D_IN, D_OUT, E, R, K = 2048, 1024, 1024, 8, 32, 2
ER = E * R
TILE_N = 256


def _fused_kernel(x_ref, wt_ref, vt_ref, ucat_ref, s_ref, bias_ref,
                  gates_ref, idx_ref, out_ref):
    xt = x_ref[...]
    xv = jnp.dot(xt, vt_ref[...], preferred_element_type=jnp.float32)

    col_e = jax.lax.broadcasted_iota(jnp.int32, (1, ER), 1) // R
    idx = idx_ref[...]
    g = gates_ref[...]
    w_full = jnp.zeros((xt.shape[0], ER), dtype=jnp.float32)
    for k in range(K):
        w_full = w_full + jnp.where(idx[:, k:k + 1] == col_e,
                                    g[:, k:k + 1], 0.0)
    coef = w_full * s_ref[...]
    t = xv * coef

    out = jnp.dot(xt, wt_ref[...], preferred_element_type=jnp.float32)
    out = out + jnp.dot(t, ucat_ref[...], preferred_element_type=jnp.float32)
    out_ref[...] = out + bias_ref[...]


@jax.jit
def kernel(x, weight_main, U_all, S_all, V_all, bias, top_k_gates,
           top_k_indices):
    wt = weight_main.T
    vt = V_all.reshape(ER, D_IN).T
    ucat = U_all.transpose(0, 2, 1).reshape(ER, D_OUT)
    s_flat = S_all.reshape(1, ER)
    bias2 = bias.reshape(1, D_OUT)

    grid = (N // TILE_N,)
    out = pl.pallas_call(
        _fused_kernel,
        grid=grid,
        in_specs=[
            pl.BlockSpec((TILE_N, D_IN), lambda i: (i, 0)),
            pl.BlockSpec((D_IN, D_OUT), lambda i: (0, 0)),
            pl.BlockSpec((D_IN, ER), lambda i: (0, 0)),
            pl.BlockSpec((ER, D_OUT), lambda i: (0, 0)),
            pl.BlockSpec((1, ER), lambda i: (0, 0)),
            pl.BlockSpec((1, D_OUT), lambda i: (0, 0)),
            pl.BlockSpec((TILE_N, K), lambda i: (i, 0)),
            pl.BlockSpec((TILE_N, K), lambda i: (i, 0)),
        ],
        out_specs=pl.BlockSpec((TILE_N, D_OUT), lambda i: (i, 0)),
        out_shape=jax.ShapeDtypeStruct((N, D_OUT), jnp.float32),
    )(x, wt, vt, ucat, s_flat, bias2, top_k_gates, top_k_indices)
    return out
